# baseline (device time: 25633 ns/iter reference)
import jax
import jax.numpy as jnp
from jax import lax
from jax.experimental import pallas as pl
from jax.experimental.pallas import tpu as pltpu

M = 512
D = 512
F = 2048
FB = F // 4
HALF = D // 2


def kernel(x, dy):
    def body(x_ref, dy_ref, out_ref,
             dyblk_ref, p_ref, snd_ref, rcv_ref, blk_ref,
             copy_sem, send_sems, recv_sems):
        mx = lax.axis_index("x")
        my = lax.axis_index("y")
        mz = lax.axis_index("z")
        q = 2 * mx + mz
        qx = 2 * (1 - mx) + mz

        bsem = pltpu.get_barrier_semaphore()
        for dev in ((1 - mx, my, mz), (mx, 1 - my, mz), (mx, my, 1 - mz)):
            pl.semaphore_signal(bsem, inc=1, device_id=dev,
                                device_id_type=pl.DeviceIdType.MESH)
        pl.semaphore_wait(bsem, 3)

        cp = pltpu.make_async_copy(dy_ref.at[:, pl.ds(q * FB, FB)],
                                   dyblk_ref, copy_sem)
        cp.start()
        cp.wait()

        p_ref[...] = lax.dot_general(
            x_ref[...].astype(jnp.bfloat16),
            dyblk_ref[...].astype(jnp.bfloat16),
            (((0,), (0,)), ((), ())),
            preferred_element_type=jnp.float32,
        )

        snd_ref[...] = p_ref[pl.ds((1 - my) * HALF, HALF), :].astype(jnp.bfloat16)
        rdma_y = pltpu.make_async_remote_copy(
            src_ref=snd_ref, dst_ref=rcv_ref,
            send_sem=send_sems.at[0], recv_sem=recv_sems.at[0],
            device_id=(mx, 1 - my, mz), device_id_type=pl.DeviceIdType.MESH)
        rdma_y.start()
        rdma_y.wait()
        blk_ref[q, :, :] = (
            p_ref[pl.ds(my * HALF, HALF), :]
            + rcv_ref[...].astype(jnp.float32)
        ).astype(jnp.bfloat16)

        rdma_x = pltpu.make_async_remote_copy(
            src_ref=blk_ref.at[q], dst_ref=blk_ref.at[q],
            send_sem=send_sems.at[1], recv_sem=recv_sems.at[1],
            device_id=(1 - mx, my, mz), device_id_type=pl.DeviceIdType.MESH)
        rdma_x.start()
        rdma_x.wait()

        rdma_z0 = pltpu.make_async_remote_copy(
            src_ref=blk_ref.at[q], dst_ref=blk_ref.at[q],
            send_sem=send_sems.at[2], recv_sem=recv_sems.at[2],
            device_id=(mx, my, 1 - mz), device_id_type=pl.DeviceIdType.MESH)
        rdma_z1 = pltpu.make_async_remote_copy(
            src_ref=blk_ref.at[qx], dst_ref=blk_ref.at[qx],
            send_sem=send_sems.at[3], recv_sem=recv_sems.at[3],
            device_id=(mx, my, 1 - mz), device_id_type=pl.DeviceIdType.MESH)
        rdma_z0.start()
        rdma_z1.start()
        rdma_z0.wait()
        rdma_z1.wait()

        for b in range(4):
            out_ref[:, b * FB:(b + 1) * FB] = blk_ref[b].astype(jnp.float32)

    return pl.pallas_call(
        body,
        out_shape=jax.ShapeDtypeStruct((HALF, F), jnp.float32),
        in_specs=[pl.BlockSpec(memory_space=pltpu.VMEM)] * 2,
        out_specs=pl.BlockSpec(memory_space=pltpu.VMEM),
        scratch_shapes=[
            pltpu.VMEM((M, FB), jnp.float32),
            pltpu.VMEM((D, FB), jnp.float32),
            pltpu.VMEM((HALF, FB), jnp.bfloat16),
            pltpu.VMEM((HALF, FB), jnp.bfloat16),
            pltpu.VMEM((4, HALF, FB), jnp.bfloat16),
            pltpu.SemaphoreType.DMA,
            pltpu.SemaphoreType.DMA((4,)),
            pltpu.SemaphoreType.DMA((4,)),
        ],
        compiler_params=pltpu.CompilerParams(collective_id=0),
    )(x, dy)


# device time: 22641 ns/iter; 1.1321x vs baseline; 1.1321x over previous
import jax
import jax.numpy as jnp
from jax import lax
from jax.experimental import pallas as pl
from jax.experimental.pallas import tpu as pltpu

M = 512
D = 512
F = 2048
FB = F // 4
HALF = D // 2


def kernel(x, dy):
    def body(x_ref, dy_ref, out_ref,
             dyblk_ref, p_ref, snd_ref, rcv_ref, blk_ref,
             copy_sem, send_sems, recv_sems):
        mx = lax.axis_index("x")
        my = lax.axis_index("y")
        mz = lax.axis_index("z")
        q = 2 * mx + mz
        qx = 2 * (1 - mx) + mz

        cp = pltpu.make_async_copy(dy_ref.at[:, pl.ds(q * FB, FB)],
                                   dyblk_ref, copy_sem)
        cp.start()

        bsem = pltpu.get_barrier_semaphore()
        for dev in ((1 - mx, my, mz), (mx, 1 - my, mz), (mx, my, 1 - mz)):
            pl.semaphore_signal(bsem, inc=1, device_id=dev,
                                device_id_type=pl.DeviceIdType.MESH)
        pl.semaphore_wait(bsem, 3)
        cp.wait()

        dyblk = dyblk_ref[...].astype(jnp.bfloat16)

        snd_ref[...] = lax.dot_general(
            x_ref[:, pl.ds((1 - my) * HALF, HALF)].astype(jnp.bfloat16),
            dyblk,
            (((0,), (0,)), ((), ())),
            preferred_element_type=jnp.float32,
        ).astype(jnp.bfloat16)

        rdma_y = pltpu.make_async_remote_copy(
            src_ref=snd_ref, dst_ref=rcv_ref,
            send_sem=send_sems.at[0], recv_sem=recv_sems.at[0],
            device_id=(mx, 1 - my, mz), device_id_type=pl.DeviceIdType.MESH)
        rdma_y.start()

        p_ref[...] = lax.dot_general(
            x_ref[:, pl.ds(my * HALF, HALF)].astype(jnp.bfloat16),
            dyblk,
            (((0,), (0,)), ((), ())),
            preferred_element_type=jnp.float32,
        )

        rdma_y.wait()
        blk_ref[q, :, :] = (
            p_ref[...] + rcv_ref[...].astype(jnp.float32)
        ).astype(jnp.bfloat16)

        rdma_x = pltpu.make_async_remote_copy(
            src_ref=blk_ref.at[q], dst_ref=blk_ref.at[q],
            send_sem=send_sems.at[1], recv_sem=recv_sems.at[1],
            device_id=(1 - mx, my, mz), device_id_type=pl.DeviceIdType.MESH)
        rdma_z0 = pltpu.make_async_remote_copy(
            src_ref=blk_ref.at[q], dst_ref=blk_ref.at[q],
            send_sem=send_sems.at[2], recv_sem=recv_sems.at[2],
            device_id=(mx, my, 1 - mz), device_id_type=pl.DeviceIdType.MESH)
        rdma_x.start()
        rdma_z0.start()
        rdma_x.wait()

        rdma_z1 = pltpu.make_async_remote_copy(
            src_ref=blk_ref.at[qx], dst_ref=blk_ref.at[qx],
            send_sem=send_sems.at[3], recv_sem=recv_sems.at[3],
            device_id=(mx, my, 1 - mz), device_id_type=pl.DeviceIdType.MESH)
        rdma_z1.start()
        rdma_z0.wait()
        rdma_z1.wait()

        for b in range(4):
            out_ref[:, b * FB:(b + 1) * FB] = blk_ref[b].astype(jnp.float32)

    return pl.pallas_call(
        body,
        out_shape=jax.ShapeDtypeStruct((HALF, F), jnp.float32),
        in_specs=[pl.BlockSpec(memory_space=pltpu.VMEM)] * 2,
        out_specs=pl.BlockSpec(memory_space=pltpu.VMEM),
        scratch_shapes=[
            pltpu.VMEM((M, FB), jnp.float32),
            pltpu.VMEM((HALF, FB), jnp.float32),
            pltpu.VMEM((HALF, FB), jnp.bfloat16),
            pltpu.VMEM((HALF, FB), jnp.bfloat16),
            pltpu.VMEM((4, HALF, FB), jnp.bfloat16),
            pltpu.SemaphoreType.DMA,
            pltpu.SemaphoreType.DMA((4,)),
            pltpu.SemaphoreType.DMA((4,)),
        ],
        compiler_params=pltpu.CompilerParams(collective_id=0),
    )(x, dy)


# device time: 18435 ns/iter; 1.3905x vs baseline; 1.2282x over previous
import jax
import jax.numpy as jnp
from jax import lax
from jax.experimental import pallas as pl
from jax.experimental.pallas import tpu as pltpu

M = 512
D = 512
F = 2048
HC = F // 2
NC = 8
CB = HC // NC
NH = 6
HALF = D // 2

MESH = pl.DeviceIdType.MESH


def kernel(x, dy):
    def body(x_ref, dy_ref, out_ref,
             dybf_ref, snd_ref, rcv_ref, p_ref, blk_ref,
             ysend, yrecv, xsend, xrecv, zsend, zrecv):
        mx = lax.axis_index("x")
        my = lax.axis_index("y")
        mz = lax.axis_index("z")
        col0 = mx * HC
        pcol0 = (1 - mx) * HC

        def gidx(j):
            return 2 * mz + j if j < 2 else j + 2

        y_dev = (mx, 1 - my, mz)
        x_dev = (1 - mx, my, mz)
        z_dev = (mx, my, 1 - mz)

        bsem = pltpu.get_barrier_semaphore()
        for dev in (x_dev, y_dev, z_dev):
            pl.semaphore_signal(bsem, inc=1, device_id=dev,
                                device_id_type=MESH)
        pl.semaphore_wait(bsem, 3)

        xt_peer = x_ref[:, pl.ds((1 - my) * HALF, HALF)].astype(jnp.bfloat16)
        xt_mine = x_ref[:, pl.ds(my * HALF, HALF)].astype(jnp.bfloat16)
        dn = (((0,), (0,)), ((), ()))

        def dyc(g):
            return dy_ref[:, pl.ds(col0 + g * CB, CB)].astype(jnp.bfloat16)

        def start_y(j):
            rdma = pltpu.make_async_remote_copy(
                src_ref=snd_ref.at[j], dst_ref=rcv_ref.at[j],
                send_sem=ysend.at[j], recv_sem=yrecv.at[j],
                device_id=y_dev, device_id_type=MESH)
            rdma.start()
            return rdma

        y_rdmas = []
        for j in range(2):
            snd_ref[j, :, :] = lax.dot_general(
                xt_peer, dyc(gidx(j)), dn,
                preferred_element_type=jnp.float32,
            ).astype(jnp.bfloat16)
            y_rdmas.append(start_y(j))
        dybf_ref[...] = dy_ref[:, pl.ds(col0, HC)].astype(jnp.bfloat16)
        for j in range(2, NH):
            g = gidx(j)
            snd_ref[j, :, :] = lax.dot_general(
                xt_peer, dybf_ref[:, g * CB:(g + 1) * CB], dn,
                preferred_element_type=jnp.float32,
            ).astype(jnp.bfloat16)
            y_rdmas.append(start_y(j))

        for j in range(NH):
            p_ref[j, :, :] = lax.dot_general(
                xt_mine, dybf_ref[:, pl.ds(gidx(j) * CB, CB)], dn,
                preferred_element_type=jnp.float32)

        x_rdmas, z_rdmas = [], []
        for j in range(NH):
            g = gidx(j)
            y_rdmas[j].wait()
            red = p_ref[j, :, :] + rcv_ref[j, :, :].astype(jnp.float32)
            blk_ref[g, :, :] = red.astype(jnp.bfloat16)
            rdma = pltpu.make_async_remote_copy(
                src_ref=blk_ref.at[g], dst_ref=blk_ref.at[NC + g],
                send_sem=xsend.at[j], recv_sem=xrecv.at[j],
                device_id=x_dev, device_id_type=MESH)
            rdma.start()
            x_rdmas.append(rdma)
            if j < 2:
                zr = pltpu.make_async_remote_copy(
                    src_ref=blk_ref.at[g], dst_ref=blk_ref.at[g],
                    send_sem=zsend.at[j], recv_sem=zrecv.at[j],
                    device_id=z_dev, device_id_type=MESH)
                zr.start()
                z_rdmas.append(zr)
            out_ref[:, pl.ds(col0 + g * CB, CB)] = red

        for j in range(NH):
            g = gidx(j)
            x_rdmas[j].wait()
            if j < 2:
                zr = pltpu.make_async_remote_copy(
                    src_ref=blk_ref.at[NC + g], dst_ref=blk_ref.at[NC + g],
                    send_sem=zsend.at[2 + j], recv_sem=zrecv.at[2 + j],
                    device_id=z_dev, device_id_type=MESH)
                zr.start()
                z_rdmas.append(zr)
            out_ref[:, pl.ds(pcol0 + g * CB, CB)] = (
                blk_ref[NC + g, :, :].astype(jnp.float32))

        for i in range(4):
            z_rdmas[i].wait()
        for i in range(2):
            gz = 2 * (1 - mz) + i
            out_ref[:, pl.ds(col0 + gz * CB, CB)] = (
                blk_ref[gz, :, :].astype(jnp.float32))
            out_ref[:, pl.ds(pcol0 + gz * CB, CB)] = (
                blk_ref[NC + gz, :, :].astype(jnp.float32))

    return pl.pallas_call(
        body,
        out_shape=jax.ShapeDtypeStruct((HALF, F), jnp.float32),
        in_specs=[pl.BlockSpec(memory_space=pltpu.VMEM)] * 2,
        out_specs=pl.BlockSpec(memory_space=pltpu.VMEM),
        scratch_shapes=[
            pltpu.VMEM((M, HC), jnp.bfloat16),
            pltpu.VMEM((NH, HALF, CB), jnp.bfloat16),
            pltpu.VMEM((NH, HALF, CB), jnp.bfloat16),
            pltpu.VMEM((NH, HALF, CB), jnp.float32),
            pltpu.VMEM((2 * NC, HALF, CB), jnp.bfloat16),
            pltpu.SemaphoreType.DMA((NH,)),
            pltpu.SemaphoreType.DMA((NH,)),
            pltpu.SemaphoreType.DMA((NH,)),
            pltpu.SemaphoreType.DMA((NH,)),
            pltpu.SemaphoreType.DMA((4,)),
            pltpu.SemaphoreType.DMA((4,)),
        ],
        compiler_params=pltpu.CompilerParams(collective_id=0),
    )(x, dy)
